# baseline (speedup 1.0000x reference)
import functools

import numpy as np
import jax
import jax.numpy as jnp
from jax import lax
from jax.experimental import pallas as pl
from jax.experimental.pallas import tpu as pltpu
from jax.experimental.pallas import tpu_sc as plsc

_B, _C, _H, _W, _M = 64, 3, 96, 320, 50
_HW = _H * _W
_NOBJ = _B * _M
_NW = 32
_CHUNK = 128
_NPAD = _NW * _CHUNK
_NSLOT = 12
_DEPTH_MEAN, _DEPTH_STD = 28.01, 16.32
_CLS0, _CLS1, _CLS2 = 1.63, 1.53, 3.88

_SLOT_ARRAY = (0, 0, 1, 1, 2, 2, 2, 3, 4, 4, 5, 5)
_SLOT_CHAN = (0, 1, 0, 1, 0, 1, 2, 0, 0, 1, 0, 1)
_ARRAY_C = (2, 2, 3, 1, 2, 2)


def _sc_gather(s2d, o2d, s3d, dep, o3d, ori, idxw):
    mesh = plsc.VectorSubcoreMesh(core_axis_name="c", subcore_axis_name="s")

    @functools.partial(
        pl.kernel,
        mesh=mesh,
        out_type=jax.ShapeDtypeStruct((_NW, _NSLOT, _CHUNK), jnp.float32),
        scratch_types=[
            pltpu.VMEM((_NSLOT, _CHUNK), jnp.int32),
            pltpu.VMEM((_NSLOT, _CHUNK), jnp.float32),
            pltpu.SemaphoreType.DMA,
        ],
    )
    def k(s2d_h, o2d_h, s3d_h, dep_h, o3d_h, ori_h, idx_h, out_h, idx_v, val_v, sem):
        wid = lax.axis_index("s") * 2 + lax.axis_index("c")
        srcs = (s2d_h, o2d_h, s3d_h, dep_h, o3d_h, ori_h)
        pltpu.sync_copy(idx_h.at[wid], idx_v)
        copies = []
        for slot in range(_NSLOT):
            src = srcs[_SLOT_ARRAY[slot]]
            copies.append(
                pltpu.async_copy(src.at[idx_v.at[slot]], val_v.at[slot], sem))
        for c in copies:
            c.wait()
        pltpu.sync_copy(val_v, out_h.at[wid])

    return k(s2d, o2d, s3d, dep, o3d, ori, idxw)


def _focal_partials(hm2d, t2d):
    rows, cols = hm2d.shape
    grid_n = 16
    br = rows // grid_n

    def body(h_ref, t_ref, pl_ref, nl_ref, np_ref):
        i = pl.program_id(0)
        x = h_ref[...]
        gt = t_ref[...]
        p = jnp.clip(jax.nn.sigmoid(x), 1e-4, 1.0 - 1e-4)
        pos = (gt == 1.0).astype(jnp.float32)
        neg = (gt < 1.0).astype(jnp.float32)
        omgt2 = jnp.square(1.0 - gt)
        nw = omgt2 * omgt2
        pls = jnp.sum(jnp.log(p) * jnp.square(1.0 - p) * pos)
        nls = jnp.sum(jnp.log(1.0 - p) * jnp.square(p) * nw * neg)
        nps = jnp.sum(pos)

        @pl.when(i == 0)
        def _():
            pl_ref[...] = pls.reshape(1, 1)
            nl_ref[...] = nls.reshape(1, 1)
            np_ref[...] = nps.reshape(1, 1)

        @pl.when(i != 0)
        def _():
            pl_ref[...] += pls.reshape(1, 1)
            nl_ref[...] += nls.reshape(1, 1)
            np_ref[...] += nps.reshape(1, 1)

    return pl.pallas_call(
        body,
        grid=(grid_n,),
        in_specs=[
            pl.BlockSpec((br, cols), lambda i: (i, 0)),
            pl.BlockSpec((br, cols), lambda i: (i, 0)),
        ],
        out_specs=[pl.BlockSpec((1, 1), lambda i: (0, 0))] * 3,
        out_shape=[jax.ShapeDtypeStruct((1, 1), jnp.float32)] * 3,
    )(hm2d, t2d)


def _atan(x):
    ax = jnp.abs(x)
    big = ax > 2.414213562373095
    mid = ax > 0.4142135623730951
    xr = jnp.where(big, -1.0 / jnp.where(big, ax, 1.0),
                   jnp.where(mid, (ax - 1.0) / (ax + 1.0), ax))
    y0 = jnp.where(big, np.pi / 2, jnp.where(mid, np.pi / 4, 0.0))
    z = xr * xr
    poly = (((8.05374449538e-2 * z - 1.38776856032e-1) * z
             + 1.99777106478e-1) * z - 3.33329491539e-1)
    r = y0 + xr + xr * z * poly
    return jnp.where(x < 0, -r, r)


def _obj_losses(gathered, aux):
    def body(g_ref, a_ref, s2_ref, o2_ref, s3_ref, po_ref, ro_ref, ms_ref):
        g = g_ref[...]
        a = a_ref[...]

        def gr(r):
            return g[r : r + 1, :]

        def ar(r):
            return a[r : r + 1, :]

        xs, ys, m = ar(0), ar(1), ar(2)
        s2d_sum = jnp.sum((jnp.abs(gr(0) - ar(3)) + jnp.abs(gr(1) - ar(4))) * m)
        o2d_sum = jnp.sum((jnp.abs(gr(2) - ar(5)) + jnp.abs(gr(3) - ar(6))) * m)
        ps0 = jnp.exp(gr(4)) * _CLS0
        ps1 = jnp.exp(gr(5)) * _CLS1
        ps2 = jnp.exp(gr(6)) * _CLS2
        s3d_sum = jnp.sum(
            (jnp.abs(ps0 - ar(7)) + jnp.abs(ps1 - ar(8)) + jnp.abs(ps2 - ar(9))) * m
        )
        dep = gr(7) * _DEPTH_STD + _DEPTH_MEAN
        px = (xs + gr(8)) * ar(14) * dep
        py = (ys + gr(9)) * ar(15) * dep
        pz = dep
        k00, k01, k02 = ar(16), ar(17), ar(18)
        k10, k11, k12 = ar(19), ar(20), ar(21)
        k20, k21, k22 = ar(22), ar(23), ar(24)
        c00 = k11 * k22 - k12 * k21
        c01 = k12 * k20 - k10 * k22
        c02 = k10 * k21 - k11 * k20
        rdet = 1.0 / (k00 * c00 + k01 * c01 + k02 * c02)
        loc0 = (c00 * px + (k02 * k21 - k01 * k22) * py + (k01 * k12 - k02 * k11) * pz) * rdet
        loc1 = (c01 * px + (k00 * k22 - k02 * k20) * py + (k02 * k10 - k00 * k12) * pz) * rdet
        loc2 = (c02 * px + (k01 * k20 - k00 * k21) * py + (k00 * k11 - k01 * k10) * pz) * rdet
        loc1 = loc1 + ps0 * 0.5
        pos_sum = jnp.sum(
            (jnp.abs(loc0 - ar(10)) + jnp.abs(loc1 - ar(11)) + jnp.abs(loc2 - ar(12))) * m
        )
        rays = _atan(loc0 / (loc2 + 1e-7))
        alphas = _atan(gr(10) / (gr(11) + 1e-7))
        alphas = jnp.where(gr(11) >= 0, alphas - np.pi / 2.0, alphas + np.pi / 2.0)
        rotys = alphas + rays
        rotys = jnp.where(rotys > np.pi, rotys - 2.0 * np.pi, rotys)
        rotys = jnp.where(rotys < -np.pi, rotys + 2.0 * np.pi, rotys)
        rot_sum = jnp.sum(jnp.abs(rotys - ar(13)) * m)
        msum = jnp.sum(m)

        s2_ref[...] = s2d_sum.reshape(1, 1)
        o2_ref[...] = o2d_sum.reshape(1, 1)
        s3_ref[...] = s3d_sum.reshape(1, 1)
        po_ref[...] = pos_sum.reshape(1, 1)
        ro_ref[...] = rot_sum.reshape(1, 1)
        ms_ref[...] = msum.reshape(1, 1)

    return pl.pallas_call(
        body,
        out_shape=[jax.ShapeDtypeStruct((1, 1), jnp.float32)] * 6,
    )(gathered, aux)


def kernel(heatmap, size_2d, offset_2d, size_3d_offset, depth, offset_3d, ori,
           t_heatmap, t_size_2d, t_offset_2d, t_size_3d_smoke, t_position,
           t_rotation_y, calibs, bbox_downsample_ratio, indices, mask_2d):
    inds = indices.reshape(-1).astype(jnp.int32)
    obj_b = jnp.arange(_NOBJ, dtype=jnp.int32) // _M

    rows = []
    for slot in range(_NSLOT):
        ca = _ARRAY_C[_SLOT_ARRAY[slot]]
        rows.append((obj_b * ca + _SLOT_CHAN[slot]) * _HW + inds)
    idx12 = jnp.pad(jnp.stack(rows), ((0, 0), (0, _NPAD - _NOBJ)))
    idxw = idx12.reshape(_NSLOT, _NW, _CHUNK).transpose(1, 0, 2)

    gat_w = _sc_gather(
        size_2d.reshape(-1), offset_2d.reshape(-1), size_3d_offset.reshape(-1),
        depth.reshape(-1), offset_3d.reshape(-1), ori.reshape(-1), idxw)
    gathered = jnp.pad(
        gat_w.transpose(1, 0, 2).reshape(_NSLOT, _NPAD), ((0, 4), (0, 0)))

    def bexp(x):
        return jnp.broadcast_to(x[:, None], (_B, _M)).reshape(-1)

    kmat = calibs[:, :3, :3]
    aux_rows = [
        (inds % _W).astype(jnp.float32),
        (inds // _W).astype(jnp.float32),
        mask_2d.reshape(-1).astype(jnp.float32),
        t_size_2d.reshape(-1, 2)[:, 0], t_size_2d.reshape(-1, 2)[:, 1],
        t_offset_2d.reshape(-1, 2)[:, 0], t_offset_2d.reshape(-1, 2)[:, 1],
        t_size_3d_smoke.reshape(-1, 3)[:, 0],
        t_size_3d_smoke.reshape(-1, 3)[:, 1],
        t_size_3d_smoke.reshape(-1, 3)[:, 2],
        t_position.reshape(-1, 3)[:, 0],
        t_position.reshape(-1, 3)[:, 1],
        t_position.reshape(-1, 3)[:, 2],
        t_rotation_y.reshape(-1),
        bexp(bbox_downsample_ratio[:, 0]), bexp(bbox_downsample_ratio[:, 1]),
        bexp(kmat[:, 0, 0]), bexp(kmat[:, 0, 1]), bexp(kmat[:, 0, 2]),
        bexp(kmat[:, 1, 0]), bexp(kmat[:, 1, 1]), bexp(kmat[:, 1, 2]),
        bexp(kmat[:, 2, 0]), bexp(kmat[:, 2, 1]), bexp(kmat[:, 2, 2]),
    ]
    aux = jnp.pad(jnp.stack(aux_rows), ((0, 32 - len(aux_rows)), (0, _NPAD - _NOBJ)))
    aux = aux.at[jnp.array([16, 20, 24]), _NOBJ:].set(1.0)

    hm2d = heatmap.reshape(-1, _W)
    t2d = t_heatmap.reshape(-1, _W)
    pls, nls, nps = _focal_partials(hm2d, t2d)

    s2s, o2s, s3s, pos_s, rot_s, msum = _obj_losses(gathered, aux)

    pls, nls, nps = pls[0, 0], nls[0, 0], nps[0, 0]
    s2s, o2s, s3s = s2s[0, 0], o2s[0, 0], s3s[0, 0]
    pos_s, rot_s, msum = pos_s[0, 0], rot_s[0, 0], msum[0, 0]

    seg = jnp.where(nps > 0, -(pls + nls) / jnp.maximum(nps, 1.0), -nls) * 5.0
    total = (seg
             + (o2s + s2s) / (msum * 2.0)
             + s3s / (msum * 3.0)
             + pos_s / (msum * 3.0)
             + rot_s / msum)
    return total


# speedup vs baseline: 1.0005x; 1.0005x over previous
import functools

import numpy as np
import jax
import jax.numpy as jnp
from jax import lax
from jax.experimental import pallas as pl
from jax.experimental.pallas import tpu as pltpu
from jax.experimental.pallas import tpu_sc as plsc

_B, _C, _H, _W, _M = 64, 3, 96, 320, 50
_HW = _H * _W
_NOBJ = _B * _M
_NW = 32
_CHUNK = 128
_NPAD = _NW * _CHUNK
_NSLOT = 12
_DEPTH_MEAN, _DEPTH_STD = 28.01, 16.32
_CLS0, _CLS1, _CLS2 = 1.63, 1.53, 3.88

_SLOT_ARRAY = (0, 0, 1, 1, 2, 2, 2, 3, 4, 4, 5, 5)
_SLOT_CHAN = (0, 1, 0, 1, 0, 1, 2, 0, 0, 1, 0, 1)
_ARRAY_C = (2, 2, 3, 1, 2, 2)


def _sc_gather(s2d, o2d, s3d, dep, o3d, ori, idxw):
    mesh = plsc.VectorSubcoreMesh(core_axis_name="c", subcore_axis_name="s")

    @functools.partial(
        pl.kernel,
        mesh=mesh,
        out_type=jax.ShapeDtypeStruct((_NW, _NSLOT, _CHUNK), jnp.float32),
        scratch_types=[
            pltpu.VMEM((_NSLOT, _CHUNK), jnp.int32),
            pltpu.VMEM((_NSLOT, _CHUNK), jnp.float32),
            pltpu.SemaphoreType.DMA,
        ],
    )
    def k(s2d_h, o2d_h, s3d_h, dep_h, o3d_h, ori_h, idx_h, out_h, idx_v, val_v, sem):
        wid = lax.axis_index("s") * 2 + lax.axis_index("c")
        srcs = (s2d_h, o2d_h, s3d_h, dep_h, o3d_h, ori_h)
        pltpu.sync_copy(idx_h.at[wid], idx_v)
        copies = []
        for slot in range(_NSLOT):
            src = srcs[_SLOT_ARRAY[slot]]
            copies.append(
                pltpu.async_copy(src.at[idx_v.at[slot]], val_v.at[slot], sem))
        for c in copies:
            c.wait()
        pltpu.sync_copy(val_v, out_h.at[wid])

    return k(s2d, o2d, s3d, dep, o3d, ori, idxw)


def _focal_partials(hm2d, t2d):
    rows, cols = hm2d.shape
    grid_n = 4
    br = rows // grid_n

    def body(h_ref, t_ref, pl_ref, nl_ref, np_ref):
        i = pl.program_id(0)
        x = h_ref[...]
        gt = t_ref[...]
        p = jnp.clip(jax.nn.sigmoid(x), 1e-4, 1.0 - 1e-4)
        pos = (gt == 1.0).astype(jnp.float32)
        neg = (gt < 1.0).astype(jnp.float32)
        omgt2 = jnp.square(1.0 - gt)
        nw = omgt2 * omgt2
        pls = jnp.sum(jnp.log(p) * jnp.square(1.0 - p) * pos)
        nls = jnp.sum(jnp.log(1.0 - p) * jnp.square(p) * nw * neg)
        nps = jnp.sum(pos)

        @pl.when(i == 0)
        def _():
            pl_ref[...] = pls.reshape(1, 1)
            nl_ref[...] = nls.reshape(1, 1)
            np_ref[...] = nps.reshape(1, 1)

        @pl.when(i != 0)
        def _():
            pl_ref[...] += pls.reshape(1, 1)
            nl_ref[...] += nls.reshape(1, 1)
            np_ref[...] += nps.reshape(1, 1)

    return pl.pallas_call(
        body,
        grid=(grid_n,),
        in_specs=[
            pl.BlockSpec((br, cols), lambda i: (i, 0)),
            pl.BlockSpec((br, cols), lambda i: (i, 0)),
        ],
        out_specs=[pl.BlockSpec((1, 1), lambda i: (0, 0))] * 3,
        out_shape=[jax.ShapeDtypeStruct((1, 1), jnp.float32)] * 3,
    )(hm2d, t2d)


def _atan(x):
    ax = jnp.abs(x)
    big = ax > 2.414213562373095
    mid = ax > 0.4142135623730951
    xr = jnp.where(big, -1.0 / jnp.where(big, ax, 1.0),
                   jnp.where(mid, (ax - 1.0) / (ax + 1.0), ax))
    y0 = jnp.where(big, np.pi / 2, jnp.where(mid, np.pi / 4, 0.0))
    z = xr * xr
    poly = (((8.05374449538e-2 * z - 1.38776856032e-1) * z
             + 1.99777106478e-1) * z - 3.33329491539e-1)
    r = y0 + xr + xr * z * poly
    return jnp.where(x < 0, -r, r)


def _obj_losses(gathered, aux):
    def body(g_ref, a_ref, s2_ref, o2_ref, s3_ref, po_ref, ro_ref, ms_ref):
        g = g_ref[...]
        a = a_ref[...]

        def gr(r):
            return g[r : r + 1, :]

        def ar(r):
            return a[r : r + 1, :]

        xs, ys, m = ar(0), ar(1), ar(2)
        s2d_sum = jnp.sum((jnp.abs(gr(0) - ar(3)) + jnp.abs(gr(1) - ar(4))) * m)
        o2d_sum = jnp.sum((jnp.abs(gr(2) - ar(5)) + jnp.abs(gr(3) - ar(6))) * m)
        ps0 = jnp.exp(gr(4)) * _CLS0
        ps1 = jnp.exp(gr(5)) * _CLS1
        ps2 = jnp.exp(gr(6)) * _CLS2
        s3d_sum = jnp.sum(
            (jnp.abs(ps0 - ar(7)) + jnp.abs(ps1 - ar(8)) + jnp.abs(ps2 - ar(9))) * m
        )
        dep = gr(7) * _DEPTH_STD + _DEPTH_MEAN
        px = (xs + gr(8)) * ar(14) * dep
        py = (ys + gr(9)) * ar(15) * dep
        pz = dep
        k00, k01, k02 = ar(16), ar(17), ar(18)
        k10, k11, k12 = ar(19), ar(20), ar(21)
        k20, k21, k22 = ar(22), ar(23), ar(24)
        c00 = k11 * k22 - k12 * k21
        c01 = k12 * k20 - k10 * k22
        c02 = k10 * k21 - k11 * k20
        rdet = 1.0 / (k00 * c00 + k01 * c01 + k02 * c02)
        loc0 = (c00 * px + (k02 * k21 - k01 * k22) * py + (k01 * k12 - k02 * k11) * pz) * rdet
        loc1 = (c01 * px + (k00 * k22 - k02 * k20) * py + (k02 * k10 - k00 * k12) * pz) * rdet
        loc2 = (c02 * px + (k01 * k20 - k00 * k21) * py + (k00 * k11 - k01 * k10) * pz) * rdet
        loc1 = loc1 + ps0 * 0.5
        pos_sum = jnp.sum(
            (jnp.abs(loc0 - ar(10)) + jnp.abs(loc1 - ar(11)) + jnp.abs(loc2 - ar(12))) * m
        )
        rays = _atan(loc0 / (loc2 + 1e-7))
        alphas = _atan(gr(10) / (gr(11) + 1e-7))
        alphas = jnp.where(gr(11) >= 0, alphas - np.pi / 2.0, alphas + np.pi / 2.0)
        rotys = alphas + rays
        rotys = jnp.where(rotys > np.pi, rotys - 2.0 * np.pi, rotys)
        rotys = jnp.where(rotys < -np.pi, rotys + 2.0 * np.pi, rotys)
        rot_sum = jnp.sum(jnp.abs(rotys - ar(13)) * m)
        msum = jnp.sum(m)

        s2_ref[...] = s2d_sum.reshape(1, 1)
        o2_ref[...] = o2d_sum.reshape(1, 1)
        s3_ref[...] = s3d_sum.reshape(1, 1)
        po_ref[...] = pos_sum.reshape(1, 1)
        ro_ref[...] = rot_sum.reshape(1, 1)
        ms_ref[...] = msum.reshape(1, 1)

    return pl.pallas_call(
        body,
        out_shape=[jax.ShapeDtypeStruct((1, 1), jnp.float32)] * 6,
    )(gathered, aux)


def kernel(heatmap, size_2d, offset_2d, size_3d_offset, depth, offset_3d, ori,
           t_heatmap, t_size_2d, t_offset_2d, t_size_3d_smoke, t_position,
           t_rotation_y, calibs, bbox_downsample_ratio, indices, mask_2d):
    inds = indices.reshape(-1).astype(jnp.int32)
    obj_b = jnp.arange(_NOBJ, dtype=jnp.int32) // _M

    rows = []
    for slot in range(_NSLOT):
        ca = _ARRAY_C[_SLOT_ARRAY[slot]]
        rows.append((obj_b * ca + _SLOT_CHAN[slot]) * _HW + inds)
    idx12 = jnp.pad(jnp.stack(rows), ((0, 0), (0, _NPAD - _NOBJ)))
    idxw = idx12.reshape(_NSLOT, _NW, _CHUNK).transpose(1, 0, 2)

    gat_w = _sc_gather(
        size_2d.reshape(-1), offset_2d.reshape(-1), size_3d_offset.reshape(-1),
        depth.reshape(-1), offset_3d.reshape(-1), ori.reshape(-1), idxw)
    gathered = jnp.pad(
        gat_w.transpose(1, 0, 2).reshape(_NSLOT, _NPAD), ((0, 4), (0, 0)))

    def bexp(x):
        return jnp.broadcast_to(x[:, None], (_B, _M)).reshape(-1)

    kmat = calibs[:, :3, :3]
    aux_rows = [
        (inds % _W).astype(jnp.float32),
        (inds // _W).astype(jnp.float32),
        mask_2d.reshape(-1).astype(jnp.float32),
        t_size_2d.reshape(-1, 2)[:, 0], t_size_2d.reshape(-1, 2)[:, 1],
        t_offset_2d.reshape(-1, 2)[:, 0], t_offset_2d.reshape(-1, 2)[:, 1],
        t_size_3d_smoke.reshape(-1, 3)[:, 0],
        t_size_3d_smoke.reshape(-1, 3)[:, 1],
        t_size_3d_smoke.reshape(-1, 3)[:, 2],
        t_position.reshape(-1, 3)[:, 0],
        t_position.reshape(-1, 3)[:, 1],
        t_position.reshape(-1, 3)[:, 2],
        t_rotation_y.reshape(-1),
        bexp(bbox_downsample_ratio[:, 0]), bexp(bbox_downsample_ratio[:, 1]),
        bexp(kmat[:, 0, 0]), bexp(kmat[:, 0, 1]), bexp(kmat[:, 0, 2]),
        bexp(kmat[:, 1, 0]), bexp(kmat[:, 1, 1]), bexp(kmat[:, 1, 2]),
        bexp(kmat[:, 2, 0]), bexp(kmat[:, 2, 1]), bexp(kmat[:, 2, 2]),
    ]
    aux = jnp.pad(jnp.stack(aux_rows), ((0, 32 - len(aux_rows)), (0, _NPAD - _NOBJ)))
    aux = aux.at[jnp.array([16, 20, 24]), _NOBJ:].set(1.0)

    hm2d = heatmap.reshape(-1, _W)
    t2d = t_heatmap.reshape(-1, _W)
    pls, nls, nps = _focal_partials(hm2d, t2d)

    s2s, o2s, s3s, pos_s, rot_s, msum = _obj_losses(gathered, aux)

    pls, nls, nps = pls[0, 0], nls[0, 0], nps[0, 0]
    s2s, o2s, s3s = s2s[0, 0], o2s[0, 0], s3s[0, 0]
    pos_s, rot_s, msum = pos_s[0, 0], rot_s[0, 0], msum[0, 0]

    seg = jnp.where(nps > 0, -(pls + nls) / jnp.maximum(nps, 1.0), -nls) * 5.0
    total = (seg
             + (o2s + s2s) / (msum * 2.0)
             + s3s / (msum * 3.0)
             + pos_s / (msum * 3.0)
             + rot_s / msum)
    return total


# speedup vs baseline: 1.0086x; 1.0081x over previous
import functools

import numpy as np
import jax
import jax.numpy as jnp
from jax import lax
from jax.experimental import pallas as pl
from jax.experimental.pallas import tpu as pltpu
from jax.experimental.pallas import tpu_sc as plsc

_B, _C, _H, _W, _M = 64, 3, 96, 320, 50
_HW = _H * _W
_NOBJ = _B * _M
_NW = 32
_CHUNK = 128
_NPAD = _NW * _CHUNK
_NSLOT = 12
_DEPTH_MEAN, _DEPTH_STD = 28.01, 16.32
_CLS0, _CLS1, _CLS2 = 1.63, 1.53, 3.88

_SLOT_ARRAY = (0, 0, 1, 1, 2, 2, 2, 3, 4, 4, 5, 5)
_SLOT_CHAN = (0, 1, 0, 1, 0, 1, 2, 0, 0, 1, 0, 1)
_ARRAY_C = (2, 2, 3, 1, 2, 2)


def _sc_gather(s2d, o2d, s3d, dep, o3d, ori, idxw):
    mesh = plsc.VectorSubcoreMesh(core_axis_name="c", subcore_axis_name="s")

    @functools.partial(
        pl.kernel,
        mesh=mesh,
        out_type=jax.ShapeDtypeStruct((_NW, _NSLOT, _CHUNK), jnp.float32),
        scratch_types=[
            pltpu.VMEM((_NSLOT, _CHUNK), jnp.int32),
            pltpu.VMEM((_NSLOT, _CHUNK), jnp.float32),
            pltpu.SemaphoreType.DMA,
        ],
    )
    def k(s2d_h, o2d_h, s3d_h, dep_h, o3d_h, ori_h, idx_h, out_h, idx_v, val_v, sem):
        wid = lax.axis_index("s") * 2 + lax.axis_index("c")
        srcs = (s2d_h, o2d_h, s3d_h, dep_h, o3d_h, ori_h)
        pltpu.sync_copy(idx_h.at[wid], idx_v)
        copies = []
        for slot in range(_NSLOT):
            src = srcs[_SLOT_ARRAY[slot]]
            copies.append(
                pltpu.async_copy(src.at[idx_v.at[slot]], val_v.at[slot], sem))
        for c in copies:
            c.wait()
        pltpu.sync_copy(val_v, out_h.at[wid])

    return k(s2d, o2d, s3d, dep, o3d, ori, idxw)


def _focal_partials(hm2d, t2d):
    rows, cols = hm2d.shape
    grid_n = 8
    br = rows // grid_n

    def body(h_ref, t_ref, pl_ref, nl_ref, np_ref):
        i = pl.program_id(0)
        x = h_ref[...]
        gt = t_ref[...]
        p = jnp.clip(jax.nn.sigmoid(x), 1e-4, 1.0 - 1e-4)
        pos = (gt == 1.0).astype(jnp.float32)
        neg = (gt < 1.0).astype(jnp.float32)
        omgt2 = jnp.square(1.0 - gt)
        nw = omgt2 * omgt2
        pls = jnp.sum(jnp.log(p) * jnp.square(1.0 - p) * pos)
        nls = jnp.sum(jnp.log(1.0 - p) * jnp.square(p) * nw * neg)
        nps = jnp.sum(pos)

        @pl.when(i == 0)
        def _():
            pl_ref[...] = pls.reshape(1, 1)
            nl_ref[...] = nls.reshape(1, 1)
            np_ref[...] = nps.reshape(1, 1)

        @pl.when(i != 0)
        def _():
            pl_ref[...] += pls.reshape(1, 1)
            nl_ref[...] += nls.reshape(1, 1)
            np_ref[...] += nps.reshape(1, 1)

    return pl.pallas_call(
        body,
        grid=(grid_n,),
        in_specs=[
            pl.BlockSpec((br, cols), lambda i: (i, 0)),
            pl.BlockSpec((br, cols), lambda i: (i, 0)),
        ],
        out_specs=[pl.BlockSpec((1, 1), lambda i: (0, 0))] * 3,
        out_shape=[jax.ShapeDtypeStruct((1, 1), jnp.float32)] * 3,
    )(hm2d, t2d)


def _atan(x):
    ax = jnp.abs(x)
    big = ax > 2.414213562373095
    mid = ax > 0.4142135623730951
    xr = jnp.where(big, -1.0 / jnp.where(big, ax, 1.0),
                   jnp.where(mid, (ax - 1.0) / (ax + 1.0), ax))
    y0 = jnp.where(big, np.pi / 2, jnp.where(mid, np.pi / 4, 0.0))
    z = xr * xr
    poly = (((8.05374449538e-2 * z - 1.38776856032e-1) * z
             + 1.99777106478e-1) * z - 3.33329491539e-1)
    r = y0 + xr + xr * z * poly
    return jnp.where(x < 0, -r, r)


def _obj_losses(gathered, aux):
    def body(g_ref, a_ref, s2_ref, o2_ref, s3_ref, po_ref, ro_ref, ms_ref):
        g = g_ref[...]
        a = a_ref[...]

        def gr(r):
            return g[r : r + 1, :]

        def ar(r):
            return a[r : r + 1, :]

        xs, ys, m = ar(0), ar(1), ar(2)
        s2d_sum = jnp.sum((jnp.abs(gr(0) - ar(3)) + jnp.abs(gr(1) - ar(4))) * m)
        o2d_sum = jnp.sum((jnp.abs(gr(2) - ar(5)) + jnp.abs(gr(3) - ar(6))) * m)
        ps0 = jnp.exp(gr(4)) * _CLS0
        ps1 = jnp.exp(gr(5)) * _CLS1
        ps2 = jnp.exp(gr(6)) * _CLS2
        s3d_sum = jnp.sum(
            (jnp.abs(ps0 - ar(7)) + jnp.abs(ps1 - ar(8)) + jnp.abs(ps2 - ar(9))) * m
        )
        dep = gr(7) * _DEPTH_STD + _DEPTH_MEAN
        px = (xs + gr(8)) * ar(14) * dep
        py = (ys + gr(9)) * ar(15) * dep
        pz = dep
        k00, k01, k02 = ar(16), ar(17), ar(18)
        k10, k11, k12 = ar(19), ar(20), ar(21)
        k20, k21, k22 = ar(22), ar(23), ar(24)
        c00 = k11 * k22 - k12 * k21
        c01 = k12 * k20 - k10 * k22
        c02 = k10 * k21 - k11 * k20
        rdet = 1.0 / (k00 * c00 + k01 * c01 + k02 * c02)
        loc0 = (c00 * px + (k02 * k21 - k01 * k22) * py + (k01 * k12 - k02 * k11) * pz) * rdet
        loc1 = (c01 * px + (k00 * k22 - k02 * k20) * py + (k02 * k10 - k00 * k12) * pz) * rdet
        loc2 = (c02 * px + (k01 * k20 - k00 * k21) * py + (k00 * k11 - k01 * k10) * pz) * rdet
        loc1 = loc1 + ps0 * 0.5
        pos_sum = jnp.sum(
            (jnp.abs(loc0 - ar(10)) + jnp.abs(loc1 - ar(11)) + jnp.abs(loc2 - ar(12))) * m
        )
        rays = _atan(loc0 / (loc2 + 1e-7))
        alphas = _atan(gr(10) / (gr(11) + 1e-7))
        alphas = jnp.where(gr(11) >= 0, alphas - np.pi / 2.0, alphas + np.pi / 2.0)
        rotys = alphas + rays
        rotys = jnp.where(rotys > np.pi, rotys - 2.0 * np.pi, rotys)
        rotys = jnp.where(rotys < -np.pi, rotys + 2.0 * np.pi, rotys)
        rot_sum = jnp.sum(jnp.abs(rotys - ar(13)) * m)
        msum = jnp.sum(m)

        s2_ref[...] = s2d_sum.reshape(1, 1)
        o2_ref[...] = o2d_sum.reshape(1, 1)
        s3_ref[...] = s3d_sum.reshape(1, 1)
        po_ref[...] = pos_sum.reshape(1, 1)
        ro_ref[...] = rot_sum.reshape(1, 1)
        ms_ref[...] = msum.reshape(1, 1)

    return pl.pallas_call(
        body,
        out_shape=[jax.ShapeDtypeStruct((1, 1), jnp.float32)] * 6,
    )(gathered, aux)


def kernel(heatmap, size_2d, offset_2d, size_3d_offset, depth, offset_3d, ori,
           t_heatmap, t_size_2d, t_offset_2d, t_size_3d_smoke, t_position,
           t_rotation_y, calibs, bbox_downsample_ratio, indices, mask_2d):
    inds = indices.reshape(-1).astype(jnp.int32)
    obj_b = jnp.arange(_NOBJ, dtype=jnp.int32) // _M

    rows = []
    for slot in range(_NSLOT):
        ca = _ARRAY_C[_SLOT_ARRAY[slot]]
        rows.append((obj_b * ca + _SLOT_CHAN[slot]) * _HW + inds)
    idx12 = jnp.pad(jnp.stack(rows), ((0, 0), (0, _NPAD - _NOBJ)))
    idxw = idx12.reshape(_NSLOT, _NW, _CHUNK).transpose(1, 0, 2)

    gat_w = _sc_gather(
        size_2d.reshape(-1), offset_2d.reshape(-1), size_3d_offset.reshape(-1),
        depth.reshape(-1), offset_3d.reshape(-1), ori.reshape(-1), idxw)
    gathered = jnp.pad(
        gat_w.transpose(1, 0, 2).reshape(_NSLOT, _NPAD), ((0, 4), (0, 0)))

    def bexp(x):
        return jnp.broadcast_to(x[:, None], (_B, _M)).reshape(-1)

    kmat = calibs[:, :3, :3]
    aux_rows = [
        (inds % _W).astype(jnp.float32),
        (inds // _W).astype(jnp.float32),
        mask_2d.reshape(-1).astype(jnp.float32),
        t_size_2d.reshape(-1, 2)[:, 0], t_size_2d.reshape(-1, 2)[:, 1],
        t_offset_2d.reshape(-1, 2)[:, 0], t_offset_2d.reshape(-1, 2)[:, 1],
        t_size_3d_smoke.reshape(-1, 3)[:, 0],
        t_size_3d_smoke.reshape(-1, 3)[:, 1],
        t_size_3d_smoke.reshape(-1, 3)[:, 2],
        t_position.reshape(-1, 3)[:, 0],
        t_position.reshape(-1, 3)[:, 1],
        t_position.reshape(-1, 3)[:, 2],
        t_rotation_y.reshape(-1),
        bexp(bbox_downsample_ratio[:, 0]), bexp(bbox_downsample_ratio[:, 1]),
        bexp(kmat[:, 0, 0]), bexp(kmat[:, 0, 1]), bexp(kmat[:, 0, 2]),
        bexp(kmat[:, 1, 0]), bexp(kmat[:, 1, 1]), bexp(kmat[:, 1, 2]),
        bexp(kmat[:, 2, 0]), bexp(kmat[:, 2, 1]), bexp(kmat[:, 2, 2]),
    ]
    aux = jnp.pad(jnp.stack(aux_rows), ((0, 32 - len(aux_rows)), (0, _NPAD - _NOBJ)))
    aux = aux.at[jnp.array([16, 20, 24]), _NOBJ:].set(1.0)

    hm2d = heatmap.reshape(-1, _W)
    t2d = t_heatmap.reshape(-1, _W)
    pls, nls, nps = _focal_partials(hm2d, t2d)

    s2s, o2s, s3s, pos_s, rot_s, msum = _obj_losses(gathered, aux)

    pls, nls, nps = pls[0, 0], nls[0, 0], nps[0, 0]
    s2s, o2s, s3s = s2s[0, 0], o2s[0, 0], s3s[0, 0]
    pos_s, rot_s, msum = pos_s[0, 0], rot_s[0, 0], msum[0, 0]

    seg = jnp.where(nps > 0, -(pls + nls) / jnp.maximum(nps, 1.0), -nls) * 5.0
    total = (seg
             + (o2s + s2s) / (msum * 2.0)
             + s3s / (msum * 3.0)
             + pos_s / (msum * 3.0)
             + rot_s / msum)
    return total
